# baseline (speedup 1.0000x reference)
import functools

import jax
import jax.numpy as jnp
from jax.experimental import pallas as pl
from jax.experimental.pallas import tpu as pltpu

K = 8
DB = 4
BLK = 128


def _sse_kernel(x_ref, W_sel_ref, b_sel_ref, W_q_ref, b_q_ref, W_k_ref,
                b_k_ref, W_v_ref, b_v_ref, W_o_ref, b_o_ref, out_ref,
                qv_s, v_s, read_s, ohs_s, w_s,
                *, B, S, d, P, c):
    f32 = jnp.float32
    bf16 = jnp.bfloat16
    BS = B * S
    i = pl.program_id(0)

    mm = functools.partial(jnp.dot, preferred_element_type=f32)

    def mm_tt(a, b):
        return jax.lax.dot_general(a, b, (((1,), (1,)), ((), ())),
                                   preferred_element_type=f32)

    def mm_00(a, b):
        return jax.lax.dot_general(a, b, (((0,), (0,)), ((), ())),
                                   preferred_element_type=f32)

    @pl.when(i == 0)
    def _routing():
        x2 = x_ref[...].reshape(BS, d)
        logitsT = jax.lax.dot_general(
            W_sel_ref[...], x2, (((0,), (1,)), ((), ())),
            preferred_element_type=f32) + b_sel_ref[...]
        kk = mm(x2, W_k_ref[...]) + b_k_ref[...]
        kmax = jnp.max(kk, axis=1, keepdims=True)
        ke = jnp.exp(kk - kmax)
        w_s[...] = ke / jnp.sum(ke, axis=1, keepdims=True)
        iota_p = jax.lax.broadcasted_iota(jnp.int32, (P, BS), 0).astype(f32)
        lg = logitsT
        neg_inf = jnp.float32(-jnp.inf)
        big = jnp.float32(P)
        for k in range(K):
            m = jnp.max(lg, axis=0, keepdims=True)
            first = jnp.min(jnp.where(lg >= m, iota_p, big), axis=0,
                            keepdims=True)
            oh = (iota_p == first)
            ohs_s[k] = oh.astype(bf16)
            lg = jnp.where(oh, neg_inf, lg)

    @pl.when(i < DB)
    def _proj():
        x2 = x_ref[...].reshape(BS, d)
        qj = mm(x2, W_q_ref[...]) + b_q_ref[...]
        vj = mm(x2, W_v_ref[...]) + b_v_ref[...]
        v_s[i] = vj
        for b in range(B):
            sl = slice(b * S, (b + 1) * S)
            contrib = mm_tt(qj[sl], vj[sl])

            @pl.when(i == 0)
            def _init():
                qv_s[b] = contrib

            @pl.when(i > 0)
            def _acc():
                qv_s[b] = qv_s[b] + contrib

    @pl.when(i == DB)
    def _attention():
        causal = (jax.lax.broadcasted_iota(jnp.int32, (S, S), 0)
                  >= jax.lax.broadcasted_iota(jnp.int32, (S, S), 1))
        cscale = causal.astype(f32) * (jnp.float32(1.0)
                                       / jnp.sqrt(jnp.float32(d)))
        w = w_s[...]
        ohs = [ohs_s[k] for k in range(K)]
        A = ohs[0]
        for k in range(1, K):
            A = A + ohs[k]
        for b in range(B):
            sl = slice(b * S, (b + 1) * S)
            wb = w[sl]
            Ab = A[:, sl]
            QVc = qv_s[b] * cscale
            masks = [mm_00(ohs[k][:, sl], Ab) for k in range(K)]
            scores = jnp.concatenate(
                [mm(masks[k] * QVc, wb) for k in range(K)], axis=1)
            smax = jnp.max(scores, axis=1, keepdims=True)
            se = jnp.exp(scores - smax)
            attn = se / jnp.sum(se, axis=1, keepdims=True)
            coeff = masks[0] * mm_tt(attn[:, 0:c], wb)
            for k in range(1, K):
                coeff = coeff + masks[k] * mm_tt(attn[:, k * c:(k + 1) * c],
                                                 wb)
            coeff = coeff * causal.astype(f32)
            vb = jnp.concatenate([v_s[j, sl] for j in range(DB)], axis=1)
            read_s[sl, :] = mm(coeff, vb)

    @pl.when(i > DB)
    def _outproj():
        out = mm(read_s[...], W_o_ref[...]) + b_o_ref[...]
        out_ref[...] = out.reshape(B, S, BLK)


def kernel(x, W_sel, b_sel, W_q, b_q, W_k, b_k, W_v, b_v, W_o, b_o):
    B, S, d = x.shape
    P = W_sel.shape[1]
    c = W_k.shape[1]
    BS = B * S
    grid_kernel = functools.partial(_sse_kernel, B=B, S=S, d=d, P=P, c=c)
    nsteps = 2 * DB + 1

    def colblk(i):
        return (0, jnp.minimum(i, DB - 1))

    def outblk(i):
        return (0, jnp.clip(i - DB - 1, 0, DB - 1))

    return pl.pallas_call(
        grid_kernel,
        grid=(nsteps,),
        in_specs=[
            pl.BlockSpec((B, S, d), lambda i: (0, 0, 0)),
            pl.BlockSpec((d, P), lambda i: (0, 0)),
            pl.BlockSpec((P, 1), lambda i: (0, 0)),
            pl.BlockSpec((d, BLK), colblk),
            pl.BlockSpec((1, BLK), colblk),
            pl.BlockSpec((d, c), lambda i: (0, 0)),
            pl.BlockSpec((1, c), lambda i: (0, 0)),
            pl.BlockSpec((d, BLK), colblk),
            pl.BlockSpec((1, BLK), colblk),
            pl.BlockSpec((d, BLK), outblk),
            pl.BlockSpec((1, BLK), outblk),
        ],
        out_specs=pl.BlockSpec((B, S, BLK), lambda i: (0, 0, outblk(i)[1])),
        out_shape=jax.ShapeDtypeStruct((B, S, d), jnp.float32),
        scratch_shapes=[
            pltpu.VMEM((B, S, S), jnp.float32),
            pltpu.VMEM((DB, BS, BLK), jnp.float32),
            pltpu.VMEM((BS, d), jnp.float32),
            pltpu.VMEM((K, P, BS), jnp.bfloat16),
            pltpu.VMEM((BS, c), jnp.float32),
        ],
    )(x, W_sel, b_sel.reshape(P, 1), W_q, b_q.reshape(1, d),
      W_k, b_k.reshape(1, c), W_v, b_v.reshape(1, d),
      W_o, b_o.reshape(1, d))


# speedup vs baseline: 1.4078x; 1.4078x over previous
import functools

import jax
import jax.numpy as jnp
from jax.experimental import pallas as pl
from jax.experimental.pallas import tpu as pltpu

K = 8


def _sse_kernel(x_ref, W_sel_ref, b_sel_ref, W_q_hbm, b_q_ref, W_k_ref,
                b_k_ref, W_v_hbm, b_v_ref, W_o_hbm, b_o_ref, out_ref,
                wq_s, wv_s, wo_s, sem_q, sem_v, sem_o,
                *, B, S, d, P, c):
    f32 = jnp.float32
    bf16 = jnp.bfloat16
    BS = B * S

    cp_q = pltpu.make_async_copy(W_q_hbm, wq_s, sem_q)
    cp_v = pltpu.make_async_copy(W_v_hbm, wv_s, sem_v)
    cp_o = pltpu.make_async_copy(W_o_hbm, wo_s, sem_o)
    cp_q.start()
    cp_v.start()
    cp_o.start()

    x = x_ref[...].reshape(BS, d)

    mm = functools.partial(jnp.dot, preferred_element_type=f32)

    def mm_tt(a, b):
        return jax.lax.dot_general(a, b, (((1,), (1,)), ((), ())),
                                   preferred_element_type=f32)

    def mm_00(a, b):
        return jax.lax.dot_general(a, b, (((0,), (0,)), ((), ())),
                                   preferred_element_type=f32)

    logitsT = jax.lax.dot_general(
        W_sel_ref[...], x, (((0,), (1,)), ((), ())),
        preferred_element_type=f32) + b_sel_ref[...]
    kk = mm(x, W_k_ref[...]) + b_k_ref[...]
    kmax = jnp.max(kk, axis=1, keepdims=True)
    ke = jnp.exp(kk - kmax)
    w = ke / jnp.sum(ke, axis=1, keepdims=True)

    iota_p = jax.lax.broadcasted_iota(jnp.int32, (P, BS), 0).astype(f32)
    lg = logitsT
    neg_inf = jnp.float32(-jnp.inf)
    big = jnp.float32(P)
    ohs = []
    for _ in range(K):
        m = jnp.max(lg, axis=0, keepdims=True)
        first = jnp.min(jnp.where(lg >= m, iota_p, big), axis=0, keepdims=True)
        oh = (iota_p == first)
        ohs.append(oh.astype(bf16))
        lg = jnp.where(oh, neg_inf, lg)
    A = ohs[0]
    for k in range(1, K):
        A = A + ohs[k]

    causal = (jax.lax.broadcasted_iota(jnp.int32, (S, S), 0)
              >= jax.lax.broadcasted_iota(jnp.int32, (S, S), 1)).astype(f32)
    inv_sqrt_d = jnp.float32(1.0) / jnp.sqrt(jnp.float32(d))

    cp_q.wait()
    q = mm(x, wq_s[...]) + b_q_ref[...]
    cp_v.wait()
    v = mm(x, wv_s[...]) + b_v_ref[...]

    reads = []
    for b in range(B):
        sl = slice(b * S, (b + 1) * S)
        qb, vb, wb = q[sl], v[sl], w[sl]
        Ab = A[:, sl]
        QVc = mm_tt(qb, vb) * (causal * inv_sqrt_d)
        masks = [mm_00(ohs[k][:, sl], Ab) for k in range(K)]
        scores = jnp.concatenate(
            [mm(masks[k] * QVc, wb) for k in range(K)], axis=1)
        smax = jnp.max(scores, axis=1, keepdims=True)
        se = jnp.exp(scores - smax)
        attn = se / jnp.sum(se, axis=1, keepdims=True)
        coeff = masks[0] * mm_tt(attn[:, 0:c], wb)
        for k in range(1, K):
            coeff = coeff + masks[k] * mm_tt(attn[:, k * c:(k + 1) * c], wb)
        coeff = coeff * causal
        reads.append(mm(coeff, vb))
    read = jnp.concatenate(reads, axis=0)
    cp_o.wait()
    out = mm(read, wo_s[...]) + b_o_ref[...]
    out_ref[...] = out.reshape(B, S, d)


def kernel(x, W_sel, b_sel, W_q, b_q, W_k, b_k, W_v, b_v, W_o, b_o):
    B, S, d = x.shape
    P = W_sel.shape[1]
    c = W_k.shape[1]
    grid_kernel = functools.partial(_sse_kernel, B=B, S=S, d=d, P=P, c=c)
    vmem = pl.BlockSpec(memory_space=pltpu.VMEM)
    hbm = pl.BlockSpec(memory_space=pltpu.MemorySpace.HBM)
    return pl.pallas_call(
        grid_kernel,
        in_specs=[vmem, vmem, vmem, hbm, vmem, vmem, vmem, hbm, vmem,
                  hbm, vmem],
        out_specs=vmem,
        out_shape=jax.ShapeDtypeStruct((B, S, d), jnp.float32),
        scratch_shapes=[
            pltpu.VMEM((d, d), jnp.float32),
            pltpu.VMEM((d, d), jnp.float32),
            pltpu.VMEM((d, d), jnp.float32),
            pltpu.SemaphoreType.DMA,
            pltpu.SemaphoreType.DMA,
            pltpu.SemaphoreType.DMA,
        ],
    )(x, W_sel, b_sel.reshape(P, 1), W_q, b_q.reshape(1, d),
      W_k, b_k.reshape(1, c), W_v, b_v.reshape(1, d),
      W_o, b_o.reshape(1, d))


# speedup vs baseline: 1.4536x; 1.0325x over previous
import functools

import jax
import jax.numpy as jnp
from jax.experimental import pallas as pl
from jax.experimental.pallas import tpu as pltpu

K = 8
_N_IN = 11


def _sse_kernel(*refs, B, S, d, P, c):
    hbm_refs = refs[:_N_IN]
    out_ref = refs[_N_IN]
    bufs = refs[_N_IN + 1:2 * _N_IN + 1]
    sems = refs[2 * _N_IN + 1:]

    copies = [pltpu.make_async_copy(h, b, s)
              for h, b, s in zip(hbm_refs, bufs, sems)]
    for cp in copies:
        cp.start()
    (cp_x, cp_wsel, cp_bsel, cp_wq, cp_bq, cp_wk, cp_bk, cp_wv, cp_bv,
     cp_wo, cp_bo) = copies
    (x_s, wsel_s, bsel_s, wq_s, bq_s, wk_s, bk_s, wv_s, bv_s,
     wo_s, bo_s) = bufs

    f32 = jnp.float32
    bf16 = jnp.bfloat16
    BS = B * S

    mm = functools.partial(jnp.dot, preferred_element_type=f32)

    def mm_tt(a, b):
        return jax.lax.dot_general(a, b, (((1,), (1,)), ((), ())),
                                   preferred_element_type=f32)

    def mm_00(a, b):
        return jax.lax.dot_general(a, b, (((0,), (0,)), ((), ())),
                                   preferred_element_type=f32)

    cp_x.wait()
    x = x_s[...].reshape(BS, d)

    cp_wsel.wait()
    cp_bsel.wait()
    logitsT = jax.lax.dot_general(
        wsel_s[...], x, (((0,), (1,)), ((), ())),
        preferred_element_type=f32) + bsel_s[...]
    cp_wk.wait()
    cp_bk.wait()
    kk = mm(x, wk_s[...]) + bk_s[...]
    kmax = jnp.max(kk, axis=1, keepdims=True)
    ke = jnp.exp(kk - kmax)
    w = ke / jnp.sum(ke, axis=1, keepdims=True)

    iota_p = jax.lax.broadcasted_iota(jnp.int32, (P, BS), 0).astype(f32)
    lg = logitsT
    neg_inf = jnp.float32(-jnp.inf)
    big = jnp.float32(P)
    ohs = []
    for _ in range(K):
        m = jnp.max(lg, axis=0, keepdims=True)
        first = jnp.min(jnp.where(lg >= m, iota_p, big), axis=0, keepdims=True)
        oh = (iota_p == first)
        ohs.append(oh.astype(bf16))
        lg = jnp.where(oh, neg_inf, lg)
    A = ohs[0]
    for k in range(1, K):
        A = A + ohs[k]

    causal = (jax.lax.broadcasted_iota(jnp.int32, (S, S), 0)
              >= jax.lax.broadcasted_iota(jnp.int32, (S, S), 1)).astype(f32)
    inv_sqrt_d = jnp.float32(1.0) / jnp.sqrt(jnp.float32(d))

    cp_wq.wait()
    cp_bq.wait()
    q = mm(x, wq_s[...]) + bq_s[...]
    cp_wv.wait()
    cp_bv.wait()
    v = mm(x, wv_s[...]) + bv_s[...]

    reads = []
    for b in range(B):
        sl = slice(b * S, (b + 1) * S)
        qb, vb, wb = q[sl], v[sl], w[sl]
        Ab = A[:, sl]
        QVc = mm_tt(qb, vb) * (causal * inv_sqrt_d)
        masks = [mm_00(ohs[k][:, sl], Ab) for k in range(K)]
        scores = jnp.concatenate(
            [mm(masks[k] * QVc, wb) for k in range(K)], axis=1)
        smax = jnp.max(scores, axis=1, keepdims=True)
        se = jnp.exp(scores - smax)
        attn = se / jnp.sum(se, axis=1, keepdims=True)
        coeff = masks[0] * mm_tt(attn[:, 0:c], wb)
        for k in range(1, K):
            coeff = coeff + masks[k] * mm_tt(attn[:, k * c:(k + 1) * c], wb)
        coeff = coeff * causal
        reads.append(mm(coeff, vb))
    read = jnp.concatenate(reads, axis=0)
    cp_wo.wait()
    cp_bo.wait()
    out = mm(read, wo_s[...]) + bo_s[...]
    out_ref[...] = out.reshape(B, S, d)


def kernel(x, W_sel, b_sel, W_q, b_q, W_k, b_k, W_v, b_v, W_o, b_o):
    B, S, d = x.shape
    P = W_sel.shape[1]
    c = W_k.shape[1]
    grid_kernel = functools.partial(_sse_kernel, B=B, S=S, d=d, P=P, c=c)
    hbm = pl.BlockSpec(memory_space=pltpu.MemorySpace.HBM)
    vmem = pl.BlockSpec(memory_space=pltpu.VMEM)
    shapes = [(B, S, d), (d, P), (P, 1), (d, d), (1, d), (d, c), (1, c),
              (d, d), (1, d), (d, d), (1, d)]
    return pl.pallas_call(
        grid_kernel,
        in_specs=[hbm] * _N_IN,
        out_specs=vmem,
        out_shape=jax.ShapeDtypeStruct((B, S, d), jnp.float32),
        scratch_shapes=(
            [pltpu.VMEM(s, jnp.float32) for s in shapes]
            + [pltpu.SemaphoreType.DMA] * _N_IN),
    )(x, W_sel, b_sel.reshape(P, 1), W_q, b_q.reshape(1, d),
      W_k, b_k.reshape(1, c), W_v, b_v.reshape(1, d),
      W_o, b_o.reshape(1, d))
